# baseline (speedup 1.0000x reference)
import functools

import jax
import jax.numpy as jnp
from jax import lax
from jax.experimental import pallas as pl
from jax.experimental.pallas import tpu as pltpu
from jax.experimental.pallas import tpu_sc as plsc

_NC = 2
_NS = 16
_NW = _NC * _NS
_SUB = 128
_G = 10


def _sc_gather3(idx0, idx1, idx2, tab0, tab1, tab2):
    n_sub = idx0.shape[0]
    d0, d1, d2 = tab0.shape[1], tab1.shape[1], tab2.shape[1]
    per_w = n_sub // _NW
    n_grp = per_w // _G
    mesh = plsc.VectorSubcoreMesh(core_axis_name="c", subcore_axis_name="s")

    @functools.partial(
        pl.kernel,
        out_type=jax.ShapeDtypeStruct((n_sub * _SUB, d0 + d1 + d2), tab0.dtype),
        mesh=mesh,
        scratch_types=[
            pltpu.VMEM((_G, _SUB), jnp.int32),
            pltpu.VMEM((_G, _SUB), jnp.int32),
            pltpu.VMEM((_G, _SUB), jnp.int32),
            pltpu.VMEM((_G * _SUB, d0), tab0.dtype),
            pltpu.VMEM((_G * _SUB, d1), tab1.dtype),
            pltpu.VMEM((_G * _SUB, d2), tab2.dtype),
            pltpu.SemaphoreType.DMA,
            pltpu.SemaphoreType.DMA,
        ],
        compiler_params=pltpu.CompilerParams(use_tc_tiling_on_sc=False),
    )
    def k(i0_hbm, i1_hbm, i2_hbm, t0_hbm, t1_hbm, t2_hbm,
          o_hbm, i0v, i1v, i2v, r0v, r1v, r2v, isem, gsem):
        wid = lax.axis_index("s") * _NC + lax.axis_index("c")

        def body(g, carry):
            base_sub = wid * per_w + g * _G
            ic = [pltpu.async_copy(i0_hbm.at[pl.ds(base_sub, _G)], i0v, isem),
                  pltpu.async_copy(i1_hbm.at[pl.ds(base_sub, _G)], i1v, isem),
                  pltpu.async_copy(i2_hbm.at[pl.ds(base_sub, _G)], i2v, isem)]
            for c in ic:
                c.wait()
            copies = []
            for j in range(_G):
                sl = pl.ds(j * _SUB, _SUB)
                copies.append(pltpu.async_copy(t0_hbm.at[i0v.at[j]], r0v.at[sl], gsem))
                copies.append(pltpu.async_copy(t1_hbm.at[i1v.at[j]], r1v.at[sl], gsem))
                copies.append(pltpu.async_copy(t2_hbm.at[i2v.at[j]], r2v.at[sl], gsem))
            for c in copies:
                c.wait()
            rows = pl.ds(base_sub * _SUB, _G * _SUB)
            pltpu.sync_copy(r0v, o_hbm.at[rows, pl.ds(0, d0)])
            pltpu.sync_copy(r1v, o_hbm.at[rows, pl.ds(d0, d1)])
            pltpu.sync_copy(r2v, o_hbm.at[rows, pl.ds(d0 + d1, d2)])
            return carry

        lax.fori_loop(0, n_grp, body, 0)

    return k(idx0, idx1, idx2, tab0, tab1, tab2)


def _tc_body(comb_ref, w_ref, b_ref, o_ref):
    o_ref[...] = jnp.dot(comb_ref[...], w_ref[...],
                         preferred_element_type=jnp.float32) + b_ref[...]


def _tc_combine(comb, w_pad, b, interpret=False):
    n = comb.shape[0]
    bs = 2048
    blk = lambda i: (i, 0)
    full = lambda i: (0, 0)
    d_out = w_pad.shape[1]
    return pl.pallas_call(
        _tc_body,
        grid=(n // bs,),
        in_specs=[
            pl.BlockSpec((bs, comb.shape[1]), blk),
            pl.BlockSpec(w_pad.shape, full),
            pl.BlockSpec((1, d_out), full),
        ],
        out_specs=pl.BlockSpec((bs, d_out), blk),
        out_shape=jax.ShapeDtypeStruct((n, d_out), jnp.float32),
        interpret=interpret,
    )(comb, w_pad, b)


def kernel(card_ids, mana_costs, card_types, powers, toughnesses,
           card_table, mana_table, type_table, power_table, tough_table, W, b):
    bsz, seq = card_ids.shape
    n = bsz * seq
    d_card = card_table.shape[1]
    n_mana, d_mana = mana_table.shape
    n_type, d_type = type_table.shape
    n_pow, d_pt = power_table.shape
    n_tgh = tough_table.shape[0]

    mt_tab = jnp.concatenate(
        [jnp.repeat(mana_table, n_type, axis=0), jnp.tile(type_table, (n_mana, 1))],
        axis=1)
    d_mt = d_mana + d_type
    d_ptp = 128 - d_card - d_mt
    pt_tab = jnp.concatenate(
        [jnp.repeat(power_table, n_tgh, axis=0), jnp.tile(tough_table, (n_pow, 1)),
         jnp.zeros((n_pow * n_tgh, d_ptp - 2 * d_pt), power_table.dtype)],
        axis=1)

    to2d = lambda a: a.reshape(n // _SUB, _SUB).astype(jnp.int32)
    card_idx = to2d(card_ids)
    mt_idx = to2d(mana_costs * n_type + card_types)
    pt_idx = to2d(powers * n_tgh + toughnesses)

    comb = _sc_gather3(card_idx, mt_idx, pt_idx,
                       card_table.astype(jnp.bfloat16),
                       mt_tab.astype(jnp.bfloat16),
                       pt_tab.astype(jnp.bfloat16))

    w_pad = jnp.concatenate(
        [W, jnp.zeros((128 - W.shape[0], W.shape[1]), W.dtype)],
        axis=0).astype(jnp.bfloat16)
    out = _tc_combine(comb, w_pad, b.reshape(1, -1))
    return out.reshape(bsz, seq, W.shape[1])


# speedup vs baseline: 1.7872x; 1.7872x over previous
import functools

import jax
import jax.numpy as jnp
from jax import lax
from jax.experimental import pallas as pl
from jax.experimental.pallas import tpu as pltpu
from jax.experimental.pallas import tpu_sc as plsc

_NC = 2
_NS = 16
_NW = _NC * _NS
_SUB = 128
_G = 4


def _sc_gather3(idx0, idx1, idx2, tab0, tab1, tab2):
    n_sub = idx0.shape[0]
    d0, d1, d2 = tab0.shape[1], tab1.shape[1], tab2.shape[1]
    per_w = n_sub // _NW
    n_grp = per_w // _G
    n_pair = n_grp // 2
    mesh = plsc.VectorSubcoreMesh(core_axis_name="c", subcore_axis_name="s")

    buf_set = [
        pltpu.VMEM((_G, _SUB), jnp.int32),
        pltpu.VMEM((_G, _SUB), jnp.int32),
        pltpu.VMEM((_G, _SUB), jnp.int32),
        pltpu.VMEM((_G * _SUB, d0), tab0.dtype),
        pltpu.VMEM((_G * _SUB, d1), tab1.dtype),
        pltpu.VMEM((_G * _SUB, d2), tab2.dtype),
        pltpu.SemaphoreType.DMA,
        pltpu.SemaphoreType.DMA,
        pltpu.SemaphoreType.DMA,
    ]

    @functools.partial(
        pl.kernel,
        out_type=jax.ShapeDtypeStruct((n_sub * _SUB, 128), tab0.dtype),
        mesh=mesh,
        scratch_types=buf_set + buf_set,
        compiler_params=pltpu.CompilerParams(use_tc_tiling_on_sc=False),
    )
    def k(i0_hbm, i1_hbm, i2_hbm, t0_hbm, t1_hbm, t2_hbm, o_hbm, *scratch):
        set_a, set_b = scratch[:9], scratch[9:]
        wid = lax.axis_index("s") * _NC + lax.axis_index("c")
        tabs = (t0_hbm, t1_hbm, t2_hbm)
        idx_hbms = (i0_hbm, i1_hbm, i2_hbm)
        lane_off = (0, d0, d0 + d1)

        def s1(g, S):
            ivs, rvs, isem, gsem = S[0:3], S[3:6], S[6], S[7]
            base_sub = wid * per_w + g * _G
            for c in [pltpu.async_copy(ih.at[pl.ds(base_sub, _G)], iv, isem)
                      for ih, iv in zip(idx_hbms, ivs)]:
                c.wait()
            for j in range(_G):
                sl = pl.ds(j * _SUB, _SUB)
                for t, iv, rv in zip(tabs, ivs, rvs):
                    pltpu.async_copy(t.at[iv.at[j]], rv.at[sl], gsem)

        def s2(g, S):
            ivs, rvs, gsem, wsem = S[0:3], S[3:6], S[7], S[8]
            for j in range(_G):
                sl = pl.ds(j * _SUB, _SUB)
                for t, iv, rv in zip(tabs, ivs, rvs):
                    pltpu.make_async_copy(t.at[iv.at[j]], rv.at[sl], gsem).wait()
            rows = pl.ds((wid * per_w + g * _G) * _SUB, _G * _SUB)
            for rv, off, d in zip(rvs, lane_off, (d0, d1, d2)):
                pltpu.async_copy(rv, o_hbm.at[rows, pl.ds(off, d)], wsem)

        def s3(S):
            rvs, wsem = S[3:6], S[8]
            rows = pl.ds(0, _G * _SUB)
            for rv, off, d in zip(rvs, lane_off, (d0, d1, d2)):
                pltpu.make_async_copy(rv, o_hbm.at[rows, pl.ds(off, d)], wsem).wait()

        s1(0, set_a)
        s1(1, set_b)

        def body(p, carry):
            g = 2 * p
            s2(g, set_a)
            s2(g + 1, set_b)
            s3(set_a)
            s1(g + 2, set_a)
            s3(set_b)
            s1(g + 3, set_b)
            return carry

        lax.fori_loop(0, n_pair - 1, body, 0)
        s2(n_grp - 2, set_a)
        s2(n_grp - 1, set_b)
        s3(set_a)
        s3(set_b)

    return k(idx0, idx1, idx2, tab0, tab1, tab2)


def _tc_body(comb_ref, w_ref, b_ref, o_ref, *, d_valid):
    x = comb_ref[...]
    lanes = lax.broadcasted_iota(jnp.int32, x.shape, 1)
    x = jnp.where(lanes < d_valid, x, 0.0)
    o_ref[...] = jnp.dot(x, w_ref[...],
                         preferred_element_type=jnp.float32) + b_ref[...]


def _tc_combine(comb, w_pad, b, d_valid, interpret=False):
    n = comb.shape[0]
    bs = 2048
    blk = lambda i: (i, 0)
    full = lambda i: (0, 0)
    d_out = w_pad.shape[1]
    return pl.pallas_call(
        functools.partial(_tc_body, d_valid=d_valid),
        grid=(n // bs,),
        in_specs=[
            pl.BlockSpec((bs, comb.shape[1]), blk),
            pl.BlockSpec(w_pad.shape, full),
            pl.BlockSpec((1, d_out), full),
        ],
        out_specs=pl.BlockSpec((bs, d_out), blk),
        out_shape=jax.ShapeDtypeStruct((n, d_out), jnp.float32),
        interpret=interpret,
    )(comb, w_pad, b)


def kernel(card_ids, mana_costs, card_types, powers, toughnesses,
           card_table, mana_table, type_table, power_table, tough_table, W, b):
    bsz, seq = card_ids.shape
    n = bsz * seq
    n_mana = mana_table.shape[0]
    n_type = type_table.shape[0]
    n_pow = power_table.shape[0]
    n_tgh = tough_table.shape[0]

    mt_tab = jnp.concatenate(
        [jnp.repeat(mana_table, n_type, axis=0), jnp.tile(type_table, (n_mana, 1))],
        axis=1)
    pt_tab = jnp.concatenate(
        [jnp.repeat(power_table, n_tgh, axis=0), jnp.tile(tough_table, (n_pow, 1))],
        axis=1)

    to2d = lambda a: a.reshape(n // _SUB, _SUB).astype(jnp.int32)
    card_idx = to2d(card_ids)
    mt_idx = to2d(mana_costs * n_type + card_types)
    pt_idx = to2d(powers * n_tgh + toughnesses)

    comb = _sc_gather3(card_idx, mt_idx, pt_idx, card_table, mt_tab, pt_tab)

    w_pad = jnp.concatenate(
        [W, jnp.zeros((128 - W.shape[0], W.shape[1]), W.dtype)], axis=0)
    out = _tc_combine(comb, w_pad, b.reshape(1, -1), W.shape[0])
    return out.reshape(bsz, seq, W.shape[1])


# speedup vs baseline: 2.0753x; 1.1612x over previous
import functools

import jax
import jax.numpy as jnp
from jax import lax
from jax.experimental import pallas as pl
from jax.experimental.pallas import tpu as pltpu
from jax.experimental.pallas import tpu_sc as plsc

_NC = 2
_NS = 16
_NW = _NC * _NS
_SUB = 128
_G = 4


def _sc_gather3(idx0, idx1, idx2, tab0, tab1, tab2):
    n_sub = idx0.shape[0]
    d0, d1, d2 = tab0.shape[1], tab1.shape[1], tab2.shape[1]
    per_w = n_sub // _NW
    n_grp = per_w // _G
    n_pair = n_grp // 2
    mesh = plsc.VectorSubcoreMesh(core_axis_name="c", subcore_axis_name="s")

    buf_set = [
        pltpu.VMEM((_G, _SUB), jnp.int32),
        pltpu.VMEM((_G, _SUB), jnp.int32),
        pltpu.VMEM((_G, _SUB), jnp.int32),
        pltpu.VMEM((_G * _SUB, d0), tab0.dtype),
        pltpu.VMEM((_G * _SUB, d1), tab1.dtype),
        pltpu.VMEM((_G * _SUB, d2), tab2.dtype),
        pltpu.SemaphoreType.DMA,
        pltpu.SemaphoreType.DMA,
        pltpu.SemaphoreType.DMA,
    ]

    @functools.partial(
        pl.kernel,
        out_type=jax.ShapeDtypeStruct((n_sub * _SUB, 128), tab0.dtype),
        mesh=mesh,
        scratch_types=buf_set + buf_set,
        compiler_params=pltpu.CompilerParams(use_tc_tiling_on_sc=False),
    )
    def k(i0_hbm, i1_hbm, i2_hbm, t0_hbm, t1_hbm, t2_hbm, o_hbm, *scratch):
        set_a, set_b = scratch[:9], scratch[9:]
        wid = lax.axis_index("s") * _NC + lax.axis_index("c")
        tabs = (t0_hbm, t1_hbm, t2_hbm)
        idx_hbms = (i0_hbm, i1_hbm, i2_hbm)
        lane_off = (0, d0, d0 + d1)

        def s1(g, S):
            ivs, rvs, isem, gsem = S[0:3], S[3:6], S[6], S[7]
            base_sub = wid * per_w + g * _G
            for c in [pltpu.async_copy(ih.at[pl.ds(base_sub, _G)], iv, isem)
                      for ih, iv in zip(idx_hbms, ivs)]:
                c.wait()
            for j in range(_G):
                sl = pl.ds(j * _SUB, _SUB)
                for t, iv, rv in zip(tabs, ivs, rvs):
                    pltpu.async_copy(t.at[iv.at[j]], rv.at[sl], gsem)

        def s2(g, S):
            ivs, rvs, gsem, wsem = S[0:3], S[3:6], S[7], S[8]
            for j in range(_G):
                sl = pl.ds(j * _SUB, _SUB)
                for t, iv, rv in zip(tabs, ivs, rvs):
                    pltpu.make_async_copy(t.at[iv.at[j]], rv.at[sl], gsem).wait()
            rows = pl.ds((wid * per_w + g * _G) * _SUB, _G * _SUB)
            for rv, off, d in zip(rvs, lane_off, (d0, d1, d2)):
                pltpu.async_copy(rv, o_hbm.at[rows, pl.ds(off, d)], wsem)

        def s3(S):
            rvs, wsem = S[3:6], S[8]
            rows = pl.ds(0, _G * _SUB)
            for rv, off, d in zip(rvs, lane_off, (d0, d1, d2)):
                pltpu.make_async_copy(rv, o_hbm.at[rows, pl.ds(off, d)], wsem).wait()

        s1(0, set_a)
        s1(1, set_b)

        def body(p, carry):
            g = 2 * p
            s2(g, set_a)
            s2(g + 1, set_b)
            s3(set_a)
            s1(g + 2, set_a)
            s3(set_b)
            s1(g + 3, set_b)
            return carry

        lax.fori_loop(0, n_pair - 1, body, 0)
        s2(n_grp - 2, set_a)
        s2(n_grp - 1, set_b)
        s3(set_a)
        s3(set_b)

    return k(idx0, idx1, idx2, tab0, tab1, tab2)


def _tc_body(comb_ref, w_ref, b_ref, o_ref, *, d_valid):
    x = comb_ref[...]
    lanes = lax.broadcasted_iota(jnp.int32, x.shape, 1)
    x = jnp.where(lanes < d_valid, x, 0.0)
    o_ref[...] = jnp.dot(x, w_ref[...],
                         preferred_element_type=jnp.float32) + b_ref[...]


def _tc_combine(comb, w_pad, b, d_valid, interpret=False):
    n = comb.shape[0]
    bs = 4096
    blk = lambda i: (i, 0)
    full = lambda i: (0, 0)
    d_out = w_pad.shape[1]
    return pl.pallas_call(
        functools.partial(_tc_body, d_valid=d_valid),
        grid=(n // bs,),
        in_specs=[
            pl.BlockSpec((bs, comb.shape[1]), blk),
            pl.BlockSpec(w_pad.shape, full),
            pl.BlockSpec((1, d_out), full),
        ],
        out_specs=pl.BlockSpec((bs, d_out), blk),
        out_shape=jax.ShapeDtypeStruct((n, d_out), jnp.float32),
        interpret=interpret,
    )(comb, w_pad, b)


def kernel(card_ids, mana_costs, card_types, powers, toughnesses,
           card_table, mana_table, type_table, power_table, tough_table, W, b):
    bsz, seq = card_ids.shape
    n = bsz * seq
    n_mana = mana_table.shape[0]
    n_type = type_table.shape[0]
    n_pow = power_table.shape[0]
    n_tgh = tough_table.shape[0]

    mt_tab = jnp.concatenate(
        [jnp.repeat(mana_table, n_type, axis=0), jnp.tile(type_table, (n_mana, 1))],
        axis=1)
    pt_tab = jnp.concatenate(
        [jnp.repeat(power_table, n_tgh, axis=0), jnp.tile(tough_table, (n_pow, 1))],
        axis=1)

    to2d = lambda a: a.reshape(n // _SUB, _SUB).astype(jnp.int32)
    card_idx = to2d(card_ids)
    mt_idx = to2d(mana_costs * n_type + card_types)
    pt_idx = to2d(powers * n_tgh + toughnesses)

    comb = _sc_gather3(card_idx, mt_idx, pt_idx, card_table, mt_tab, pt_tab)

    w_pad = jnp.concatenate(
        [W, jnp.zeros((128 - W.shape[0], W.shape[1]), W.dtype)], axis=0)
    out = _tc_combine(comb, w_pad, b.reshape(1, -1), W.shape[0])
    return out.reshape(bsz, seq, W.shape[1])


# speedup vs baseline: 2.1876x; 1.0541x over previous
import functools

import jax
import jax.numpy as jnp
from jax import lax
from jax.experimental import pallas as pl
from jax.experimental.pallas import tpu as pltpu
from jax.experimental.pallas import tpu_sc as plsc

_NC = 2
_NS = 16
_NW = _NC * _NS
_SUB = 128
_G = 4


def _sc_gather3(idx0, idx1, idx2, tab0, tab1, tab2):
    n_sub = idx0.shape[0]
    d0, d1, d2 = tab0.shape[1], tab1.shape[1], tab2.shape[1]
    per_w = n_sub // _NW
    n_grp = per_w // _G
    n_pair = n_grp // 2
    mesh = plsc.VectorSubcoreMesh(core_axis_name="c", subcore_axis_name="s")

    buf_set = [
        pltpu.VMEM((_G, _SUB), jnp.int32),
        pltpu.VMEM((_G, _SUB), jnp.int32),
        pltpu.VMEM((_G, _SUB), jnp.int32),
        pltpu.VMEM((_G * _SUB, d0), tab0.dtype),
        pltpu.VMEM((_G * _SUB, d1), tab1.dtype),
        pltpu.VMEM((_G * _SUB, d2), tab2.dtype),
        pltpu.SemaphoreType.DMA,
        pltpu.SemaphoreType.DMA,
        pltpu.SemaphoreType.DMA,
    ]

    @functools.partial(
        pl.kernel,
        out_type=jax.ShapeDtypeStruct((n_sub * _SUB, 128), tab0.dtype),
        mesh=mesh,
        scratch_types=buf_set + buf_set,
        compiler_params=pltpu.CompilerParams(use_tc_tiling_on_sc=False),
    )
    def k(i0_hbm, i1_hbm, i2_hbm, t0_hbm, t1_hbm, t2_hbm, o_hbm, *scratch):
        set_a, set_b = scratch[:9], scratch[9:]
        wid = lax.axis_index("s") * _NC + lax.axis_index("c")
        tabs = (t0_hbm, t1_hbm, t2_hbm)
        idx_hbms = (i0_hbm, i1_hbm, i2_hbm)
        lane_off = (0, d0, d0 + d1)

        def s1(g, S):
            ivs, rvs, isem, gsem = S[0:3], S[3:6], S[6], S[7]
            base_sub = wid * per_w + g * _G
            for c in [pltpu.async_copy(ih.at[pl.ds(base_sub, _G)], iv, isem)
                      for ih, iv in zip(idx_hbms, ivs)]:
                c.wait()
            for j in range(_G):
                sl = pl.ds(j * _SUB, _SUB)
                for t, iv, rv in zip(tabs, ivs, rvs):
                    pltpu.async_copy(t.at[iv.at[j]], rv.at[sl], gsem)

        def s2(g, S):
            ivs, rvs, gsem, wsem = S[0:3], S[3:6], S[7], S[8]
            for j in range(_G):
                sl = pl.ds(j * _SUB, _SUB)
                for t, iv, rv in zip(tabs, ivs, rvs):
                    pltpu.make_async_copy(t.at[iv.at[j]], rv.at[sl], gsem).wait()
            rows = pl.ds((wid * per_w + g * _G) * _SUB, _G * _SUB)
            for rv, off, d in zip(rvs, lane_off, (d0, d1, d2)):
                pltpu.async_copy(rv, o_hbm.at[rows, pl.ds(off, d)], wsem)

        def s3(S):
            rvs, wsem = S[3:6], S[8]
            rows = pl.ds(0, _G * _SUB)
            for rv, off, d in zip(rvs, lane_off, (d0, d1, d2)):
                pltpu.make_async_copy(rv, o_hbm.at[rows, pl.ds(off, d)], wsem).wait()

        s1(0, set_a)
        s1(1, set_b)

        def body(p, carry):
            g = 2 * p
            s2(g, set_a)
            s2(g + 1, set_b)
            s3(set_a)
            s1(g + 2, set_a)
            s3(set_b)
            s1(g + 3, set_b)
            return carry

        lax.fori_loop(0, n_pair - 1, body, 0)
        s2(n_grp - 2, set_a)
        s2(n_grp - 1, set_b)
        s3(set_a)
        s3(set_b)

    return k(idx0, idx1, idx2, tab0, tab1, tab2)


def _tc_body(comb_ref, w_ref, b_ref, o_ref, *, d_valid):
    x = comb_ref[...]
    lanes = lax.broadcasted_iota(jnp.int32, x.shape, 1)
    x = jnp.where(lanes < d_valid, x, 0.0)
    o_ref[...] = jnp.dot(x, w_ref[...],
                         preferred_element_type=jnp.float32) + b_ref[...]


def _tc_combine(comb, w_pad, b, d_valid, interpret=False):
    n = comb.shape[0]
    bs = 8192
    blk = lambda i: (i, 0)
    full = lambda i: (0, 0)
    d_out = w_pad.shape[1]
    return pl.pallas_call(
        functools.partial(_tc_body, d_valid=d_valid),
        grid=(n // bs,),
        in_specs=[
            pl.BlockSpec((bs, comb.shape[1]), blk),
            pl.BlockSpec(w_pad.shape, full),
            pl.BlockSpec((1, d_out), full),
        ],
        out_specs=pl.BlockSpec((bs, d_out), blk),
        out_shape=jax.ShapeDtypeStruct((n, d_out), jnp.float32),
        interpret=interpret,
    )(comb, w_pad, b)


def kernel(card_ids, mana_costs, card_types, powers, toughnesses,
           card_table, mana_table, type_table, power_table, tough_table, W, b):
    bsz, seq = card_ids.shape
    n = bsz * seq
    n_mana = mana_table.shape[0]
    n_type = type_table.shape[0]
    n_pow = power_table.shape[0]
    n_tgh = tough_table.shape[0]

    mt_tab = jnp.concatenate(
        [jnp.repeat(mana_table, n_type, axis=0), jnp.tile(type_table, (n_mana, 1))],
        axis=1)
    pt_tab = jnp.concatenate(
        [jnp.repeat(power_table, n_tgh, axis=0), jnp.tile(tough_table, (n_pow, 1))],
        axis=1)

    to2d = lambda a: a.reshape(n // _SUB, _SUB).astype(jnp.int32)
    card_idx = to2d(card_ids)
    mt_idx = to2d(mana_costs * n_type + card_types)
    pt_idx = to2d(powers * n_tgh + toughnesses)

    comb = _sc_gather3(card_idx, mt_idx, pt_idx, card_table, mt_tab, pt_tab)

    w_pad = jnp.concatenate(
        [W, jnp.zeros((128 - W.shape[0], W.shape[1]), W.dtype)], axis=0)
    out = _tc_combine(comb, w_pad, b.reshape(1, -1), W.shape[0])
    return out.reshape(bsz, seq, W.shape[1])


# speedup vs baseline: 2.2000x; 1.0057x over previous
import functools

import jax
import jax.numpy as jnp
from jax import lax
from jax.experimental import pallas as pl
from jax.experimental.pallas import tpu as pltpu
from jax.experimental.pallas import tpu_sc as plsc

_NC = 2
_NS = 16
_NW = _NC * _NS
_SUB = 128
_G = 4


def _sc_gather3(idx0, idx1, idx2, tab0, tab1, tab2):
    n_sub = idx0.shape[0]
    d0, d1, d2 = tab0.shape[1], tab1.shape[1], tab2.shape[1]
    per_w = n_sub // _NW
    n_grp = per_w // _G
    n_pair = n_grp // 2
    mesh = plsc.VectorSubcoreMesh(core_axis_name="c", subcore_axis_name="s")

    buf_set = [
        pltpu.VMEM((_G, _SUB), jnp.int32),
        pltpu.VMEM((_G, _SUB), jnp.int32),
        pltpu.VMEM((_G, _SUB), jnp.int32),
        pltpu.VMEM((_G * _SUB, d0), tab0.dtype),
        pltpu.VMEM((_G * _SUB, d1), tab1.dtype),
        pltpu.VMEM((_G * _SUB, d2), tab2.dtype),
        pltpu.SemaphoreType.DMA,
        pltpu.SemaphoreType.DMA,
        pltpu.SemaphoreType.DMA,
    ]

    @functools.partial(
        pl.kernel,
        out_type=jax.ShapeDtypeStruct((n_sub * _SUB, 128), tab0.dtype),
        mesh=mesh,
        scratch_types=buf_set + buf_set,
        compiler_params=pltpu.CompilerParams(use_tc_tiling_on_sc=False),
    )
    def k(i0_hbm, i1_hbm, i2_hbm, t0_hbm, t1_hbm, t2_hbm, o_hbm, *scratch):
        set_a, set_b = scratch[:9], scratch[9:]
        wid = lax.axis_index("s") * _NC + lax.axis_index("c")
        tabs = (t0_hbm, t1_hbm, t2_hbm)
        idx_hbms = (i0_hbm, i1_hbm, i2_hbm)
        lane_off = (0, d0, d0 + d1)

        def s1(g, S):
            ivs, rvs, isem, gsem = S[0:3], S[3:6], S[6], S[7]
            base_sub = wid * per_w + g * _G
            for c in [pltpu.async_copy(ih.at[pl.ds(base_sub, _G)], iv, isem)
                      for ih, iv in zip(idx_hbms, ivs)]:
                c.wait()
            for j in range(_G):
                sl = pl.ds(j * _SUB, _SUB)
                for t, iv, rv in zip(tabs, ivs, rvs):
                    pltpu.async_copy(t.at[iv.at[j]], rv.at[sl], gsem)

        def s2(g, S):
            ivs, rvs, gsem, wsem = S[0:3], S[3:6], S[7], S[8]
            for j in range(_G):
                sl = pl.ds(j * _SUB, _SUB)
                for t, iv, rv in zip(tabs, ivs, rvs):
                    pltpu.make_async_copy(t.at[iv.at[j]], rv.at[sl], gsem).wait()
            rows = pl.ds((wid * per_w + g * _G) * _SUB, _G * _SUB)
            for rv, off, d in zip(rvs, lane_off, (d0, d1, d2)):
                pltpu.async_copy(rv, o_hbm.at[rows, pl.ds(off, d)], wsem)

        def s3(S):
            rvs, wsem = S[3:6], S[8]
            rows = pl.ds(0, _G * _SUB)
            for rv, off, d in zip(rvs, lane_off, (d0, d1, d2)):
                pltpu.make_async_copy(rv, o_hbm.at[rows, pl.ds(off, d)], wsem).wait()

        s1(0, set_a)
        s1(1, set_b)

        def body(p, carry):
            g = 2 * p
            s2(g, set_a)
            s2(g + 1, set_b)
            s3(set_a)
            s1(g + 2, set_a)
            s3(set_b)
            s1(g + 3, set_b)
            return carry

        lax.fori_loop(0, n_pair - 1, body, 0)
        s2(n_grp - 2, set_a)
        s2(n_grp - 1, set_b)
        s3(set_a)
        s3(set_b)

    return k(idx0, idx1, idx2, tab0, tab1, tab2)


def _tc_body(comb_ref, w_ref, b_ref, o_ref, *, d_valid):
    x = comb_ref[...]
    lanes = lax.broadcasted_iota(jnp.int32, x.shape, 1)
    x = jnp.where(lanes < d_valid, x, 0.0)
    o_ref[...] = jnp.dot(x, w_ref[...],
                         preferred_element_type=jnp.float32) + b_ref[...]


def _tc_combine(comb, w_pad, b, d_valid, interpret=False):
    n = comb.shape[0]
    bs = 16384
    blk = lambda i: (i, 0)
    full = lambda i: (0, 0)
    d_out = w_pad.shape[1]
    return pl.pallas_call(
        functools.partial(_tc_body, d_valid=d_valid),
        grid=(n // bs,),
        in_specs=[
            pl.BlockSpec((bs, comb.shape[1]), blk),
            pl.BlockSpec(w_pad.shape, full),
            pl.BlockSpec((1, d_out), full),
        ],
        out_specs=pl.BlockSpec((bs, d_out), blk),
        out_shape=jax.ShapeDtypeStruct((n, d_out), jnp.float32),
        interpret=interpret,
    )(comb, w_pad, b)


def kernel(card_ids, mana_costs, card_types, powers, toughnesses,
           card_table, mana_table, type_table, power_table, tough_table, W, b):
    bsz, seq = card_ids.shape
    n = bsz * seq
    n_mana = mana_table.shape[0]
    n_type = type_table.shape[0]
    n_pow = power_table.shape[0]
    n_tgh = tough_table.shape[0]

    mt_tab = jnp.concatenate(
        [jnp.repeat(mana_table, n_type, axis=0), jnp.tile(type_table, (n_mana, 1))],
        axis=1)
    pt_tab = jnp.concatenate(
        [jnp.repeat(power_table, n_tgh, axis=0), jnp.tile(tough_table, (n_pow, 1))],
        axis=1)

    to2d = lambda a: a.reshape(n // _SUB, _SUB).astype(jnp.int32)
    card_idx = to2d(card_ids)
    mt_idx = to2d(mana_costs * n_type + card_types)
    pt_idx = to2d(powers * n_tgh + toughnesses)

    comb = _sc_gather3(card_idx, mt_idx, pt_idx, card_table, mt_tab, pt_tab)

    w_pad = jnp.concatenate(
        [W, jnp.zeros((128 - W.shape[0], W.shape[1]), W.dtype)], axis=0)
    out = _tc_combine(comb, w_pad, b.reshape(1, -1), W.shape[0])
    return out.reshape(bsz, seq, W.shape[1])
